# baseline (speedup 1.0000x reference)
import functools

import jax
import jax.numpy as jnp
from jax import lax
from jax.experimental import pallas as pl
from jax.experimental.pallas import tpu as pltpu
from jax.experimental.pallas import tpu_sc as plsc

_BATCH = 16384
_D = 64
_LANES = 16

_info = plsc.get_sparse_core_info()
_NC, _NS = _info.num_cores, _info.num_subcores
_NW = _NC * _NS
_BPW = _BATCH // _NW
_CHUNK = 128
_NCHUNK = _BPW // _CHUNK
_NGROUP = _BPW // _LANES


def _body(user_hbm, item_hbm, uf_hbm, if_hbm, out_hbm,
          idx_u, idx_i, rows_u, rows_i, out_v, sem):
    wid = lax.axis_index("s") * _NC + lax.axis_index("c")
    base = wid * _BPW

    for c in range(_NCHUNK):
        pltpu.sync_copy(user_hbm.at[pl.ds(base + c * _CHUNK, _CHUNK)],
                        idx_u.at[c])
        pltpu.sync_copy(item_hbm.at[pl.ds(base + c * _CHUNK, _CHUNK)],
                        idx_i.at[c])

    copies = []
    for c in range(_NCHUNK):
        copies.append(pltpu.async_copy(
            uf_hbm.at[idx_u.at[c]], rows_u.at[pl.ds(c * _CHUNK, _CHUNK)], sem))
        copies.append(pltpu.async_copy(
            if_hbm.at[idx_i.at[c]], rows_i.at[pl.ds(c * _CHUNK, _CHUNK)], sem))
    for cp in copies:
        cp.wait()

    iota = lax.iota(jnp.int32, _LANES)

    def group(g, carry):
        row = g * _LANES + iota
        acc = jnp.zeros((_LANES,), jnp.float32)
        for d in range(_D):
            col = jnp.full((_LANES,), d, jnp.int32)
            u = plsc.load_gather(rows_u, [row, col])
            v = plsc.load_gather(rows_i, [row, col])
            acc = acc + u * v
        out_v[pl.ds(g * _LANES, _LANES)] = acc
        return carry

    lax.fori_loop(0, _NGROUP, group, 0)

    pltpu.sync_copy(out_v, out_hbm.at[pl.ds(base, _BPW)])


@jax.jit
def _run(user, item, user_factors, item_factors):
    mesh = plsc.VectorSubcoreMesh(core_axis_name="c", subcore_axis_name="s")
    fn = pl.kernel(
        _body,
        mesh=mesh,
        out_type=jax.ShapeDtypeStruct((_BATCH,), jnp.float32),
        scratch_types=[
            pltpu.VMEM((_NCHUNK, _CHUNK), jnp.int32),
            pltpu.VMEM((_NCHUNK, _CHUNK), jnp.int32),
            pltpu.VMEM((_BPW, _D), jnp.float32),
            pltpu.VMEM((_BPW, _D), jnp.float32),
            pltpu.VMEM((_BPW,), jnp.float32),
            pltpu.SemaphoreType.DMA,
        ],
        compiler_params=pltpu.CompilerParams(
            needs_layout_passes=False, use_tc_tiling_on_sc=False),
    )
    return fn(user, item, user_factors, item_factors)


def kernel(user, item, user_factors, item_factors):
    return _run(user, item, user_factors, item_factors)
